# Initial kernel scaffold; baseline (speedup 1.0000x reference)
import jax
import jax.numpy as jnp
from jax.experimental import pallas as pl


def kernel(x, psm_single, record_len, pairwise_t_matrix, trajectory, gauss_kernel):
    raise NotImplementedError("write your pallas kernel here")


# speedup vs baseline: 354.1120x; 354.1120x over previous
import functools

import jax
import jax.numpy as jnp
from jax.experimental import pallas as pl

_THRESHOLD = 0.5


def _mask_kernel(psm_ref, gk_ref, mask_ref, rate_ref):
    p = psm_ref[...]
    m = jnp.max(jax.nn.sigmoid(p), axis=1)
    L, H, W = m.shape
    mp = jnp.pad(m, ((0, 0), (2, 2), (2, 2)))
    sm = jnp.zeros((L, H, W), dtype=jnp.float32)
    for i in range(5):
        for j in range(5):
            sm = sm + gk_ref[i, j] * jax.lax.slice(mp, (0, i, j), (L, i + H, j + W))
    msk = jnp.where(sm > _THRESHOLD, 1.0, 0.0).astype(jnp.float32)
    rate_ref[...] = (jnp.sum(msk) / (L * H * W)).reshape(1, 1)
    cav = jax.lax.broadcasted_iota(jnp.int32, (L, H, W), 0)
    mask_ref[...] = jnp.where(cav == 0, 1.0, msk)


def _fuse_kernel(x_ref, mask_ref, out_ref):
    inv_sqrt_c = 1.0 / jnp.sqrt(jnp.float32(x_ref.shape[1]))
    x0 = x_ref[0]
    s = [jnp.sum(x0 * x0, axis=0) * inv_sqrt_c]
    for m in range(1, x_ref.shape[0]):
        dot = jnp.sum(x0 * x_ref[m], axis=0) * inv_sqrt_c
        s.append(mask_ref[m] * dot)
    smax = s[0]
    for m in range(1, len(s)):
        smax = jnp.maximum(smax, s[m])
    e = [jnp.exp(v - smax) for v in s]
    denom = e[0]
    for m in range(1, len(e)):
        denom = denom + e[m]
    inv = 1.0 / denom
    out = (e[0] * inv)[None] * x0
    for m in range(1, len(e)):
        out = out + (e[m] * inv * mask_ref[m])[None] * x_ref[m]
    out_ref[...] = out


def kernel(x, psm_single, record_len, pairwise_t_matrix, trajectory, gauss_kernel):
    N, C, H, W = x.shape
    L = psm_single.shape[0]

    mask, rate = pl.pallas_call(
        _mask_kernel,
        out_shape=(
            jax.ShapeDtypeStruct((L, H, W), jnp.float32),
            jax.ShapeDtypeStruct((1, 1), jnp.float32),
        ),
    )(psm_single, gauss_kernel)

    ROWS = 8
    grid = (H // ROWS,)
    x_fuse = pl.pallas_call(
        _fuse_kernel,
        grid=grid,
        in_specs=[
            pl.BlockSpec((N, C, ROWS, W), lambda i: (0, 0, i, 0)),
            pl.BlockSpec((N, ROWS, W), lambda i: (0, i, 0)),
        ],
        out_specs=pl.BlockSpec((C, ROWS, W), lambda i: (0, i, 0)),
        out_shape=jax.ShapeDtypeStruct((C, H, W), jnp.float32),
    )(x, mask)

    B = pairwise_t_matrix.shape[0]
    comm_rates = rate.reshape(()) / B
    return x_fuse[None], comm_rates
